# baseline (speedup 1.0000x reference)
import functools

import jax
import jax.numpy as jnp
from jax import lax
from jax.experimental import pallas as pl
from jax.experimental.pallas import tpu as pltpu
from jax.experimental.pallas import tpu_sc as plsc

VOCAB = 1000000
EMBED_DIM = 64
BATCH = 16384
HIST_LEN = 50

_B = BATCH * HIST_LEN
_NW = 32
_NBLK = _B // 128
_BLK_PER_W = _NBLK // _NW
_GBUF = 4
_SBUF = 2

_mesh = plsc.VectorSubcoreMesh(core_axis_name="c", subcore_axis_name="s")


@functools.partial(
    pl.kernel,
    mesh=_mesh,
    out_type=jax.ShapeDtypeStruct((HIST_LEN, 8, 128, 8, 128), jnp.float32),
    scratch_types=[
        pltpu.VMEM((_BLK_PER_W, 128), jnp.int32),
        pltpu.VMEM((_GBUF, 128), jnp.int32),
        pltpu.VMEM((_GBUF, 128), jnp.int32),
        [pltpu.VMEM((128, 128), jnp.float32) for _ in range(_GBUF)],
        [pltpu.VMEM((64, 128), jnp.float32) for _ in range(_SBUF)],
        [pltpu.SemaphoreType.DMA for _ in range(_GBUF)],
        [pltpu.SemaphoreType.DMA for _ in range(_SBUF)],
    ],
    compiler_params=pltpu.CompilerParams(
        use_tc_tiling_on_sc=True, needs_layout_passes=False),
)
def _lookup_kernel(table_hbm, idx_hbm, out_hbm, idx_v, sr_v, colb_v,
                   staged, tout, gsems, ssems):
    wid = lax.axis_index("s") * 2 + lax.axis_index("c")
    blk0 = wid * _BLK_PER_W
    pltpu.sync_copy(idx_hbm.at[pl.ds(blk0, _BLK_PER_W)], idx_v)

    iotas = [lax.iota(jnp.int32, 16) + (16 * g) for g in range(8)]

    def prep(t, p):
        for g in range(8):
            v = idx_v[t, pl.ds(16 * g, 16)]
            sr_v[p, pl.ds(16 * g, 16)] = lax.shift_right_logical(v, 1)
            colb_v[p, pl.ds(16 * g, 16)] = lax.shift_left(
                lax.bitwise_and(v, 1), 6)

    def fire_gather(p):
        pltpu.async_copy(table_hbm.at[sr_v.at[p]], staged[p], gsems[p])

    def wait_gather(p):
        pltpu.make_async_copy(table_hbm.at[pl.ds(0, 128)], staged[p],
                              gsems[p]).wait()

    def wait_stores(q):
        pltpu.make_async_copy(table_hbm.at[pl.ds(0, 32)], tout[q],
                              ssems[q]).wait()

    def transpose(p, q):
        colbs = [colb_v[p, pl.ds(16 * g, 16)] for g in range(8)]

        def frow(i, carry):
            f0 = i * 4
            for df in range(4):
                for g in range(8):
                    vals = plsc.load_gather(
                        staged[p], [iotas[g], colbs[g] + (f0 + df)])
                    tout[q][f0 + df, pl.ds(16 * g, 16)] = vals
            return carry

        lax.fori_loop(0, 16, frow, 0)

    def fire_stores(t, q):
        blk = blk0 + t
        h = blk // 128
        bg = lax.rem(blk, 128)
        for fg in range(8):
            pltpu.async_copy(tout[q].at[pl.ds(8 * fg, 8)],
                             out_hbm.at[h, fg, bg], ssems[q])

    def body(t, p, q, first_stores, last_group):
        wait_gather(p)
        if not first_stores:
            wait_stores(q)
        transpose(p, q)
        fire_stores(t, q)
        if not last_group:
            prep(t + _GBUF, p)
            fire_gather(p)

    for p in range(_GBUF):
        prep(p, p)
        fire_gather(p)

    for p in range(_GBUF):
        body(p, p, p % _SBUF, first_stores=(p < _SBUF), last_group=False)

    def outer(g, carry):
        t0 = g * _GBUF
        for p in range(_GBUF):
            body(t0 + p, p, p % _SBUF, False, False)
        return carry

    lax.fori_loop(1, _BLK_PER_W // _GBUF - 1, outer, 0, unroll=False)

    for p in range(_GBUF):
        t = _BLK_PER_W - _GBUF + p
        body(t, p, t % _SBUF, first_stores=False, last_group=True)
    for q in range(_SBUF):
        wait_stores(q)


def kernel(item_embeddings, batch_data):
    table = item_embeddings.reshape(VOCAB // 2, 2 * EMBED_DIM)
    idx = batch_data.T.astype(jnp.int32).reshape(_NBLK, 128)
    out5d = _lookup_kernel(table, idx)
    out = out5d.transpose(2, 4, 0, 1, 3).reshape(BATCH, HIST_LEN, EMBED_DIM)
    return out


# speedup vs baseline: 1.4698x; 1.4698x over previous
import functools

import jax
import jax.numpy as jnp
from jax import lax
from jax.experimental import pallas as pl
from jax.experimental.pallas import tpu as pltpu
from jax.experimental.pallas import tpu_sc as plsc

VOCAB = 1000000
EMBED_DIM = 64
BATCH = 16384
HIST_LEN = 50

_B = BATCH * HIST_LEN
_NW = 32
_BPW = _B // _NW
_CHUNK = 128
_GPS = 5
_STEP = _CHUNK * _GPS
_NSTEPS = _BPW // _STEP
_NBUF = 2
_NOUTER = _NSTEPS // _NBUF
_ROWS_PER_W = _BPW // _CHUNK

_mesh = plsc.VectorSubcoreMesh(core_axis_name="c", subcore_axis_name="s")


@functools.partial(
    pl.kernel,
    mesh=_mesh,
    out_type=jax.ShapeDtypeStruct((_B, EMBED_DIM), jnp.float32),
    scratch_types=[
        pltpu.VMEM((_ROWS_PER_W, _CHUNK), jnp.int32),
        [pltpu.VMEM((_STEP, EMBED_DIM), jnp.float32) for _ in range(_NBUF)],
        [pltpu.SemaphoreType.DMA for _ in range(_NBUF)],
    ],
    compiler_params=pltpu.CompilerParams(use_tc_tiling_on_sc=False),
)
def _gather_kernel(table_hbm, idx_hbm, out_hbm, idx_v, rows_bufs, sems):
    wid = lax.axis_index("s") * 2 + lax.axis_index("c")
    base = wid * _BPW
    pltpu.sync_copy(idx_hbm.at[pl.ds(wid * _ROWS_PER_W, _ROWS_PER_W)], idx_v)

    def fire(step, b):
        for i in range(_GPS):
            pltpu.async_copy(
                table_hbm.at[idx_v.at[step * _GPS + i]],
                rows_bufs[b].at[pl.ds(i * _CHUNK, _CHUNK)],
                sems[b],
            )

    def drain_and_store(step, b):
        pltpu.make_async_copy(
            table_hbm.at[pl.ds(0, _STEP)], rows_bufs[b], sems[b]
        ).wait()
        pltpu.sync_copy(rows_bufs[b], out_hbm.at[pl.ds(base + step * _STEP, _STEP)])

    for b in range(_NBUF):
        fire(b, b)

    def outer(t, carry):
        for b in range(_NBUF):
            step = t * _NBUF + b
            drain_and_store(step, b)
            fire(step + _NBUF, b)
        return carry

    lax.fori_loop(0, _NOUTER - 1, outer, 0)

    for b in range(_NBUF):
        drain_and_store((_NOUTER - 1) * _NBUF + b, b)


def kernel(item_embeddings, batch_data):
    idx = batch_data.reshape(-1).astype(jnp.int32)
    idx2d = idx.reshape(_B // _CHUNK, _CHUNK)
    out = _gather_kernel(item_embeddings, idx2d)
    return out.reshape(BATCH, HIST_LEN, EMBED_DIM)
